# baseline (speedup 1.0000x reference)
import functools

import jax
import jax.numpy as jnp
from jax import lax
from jax.experimental import pallas as pl
from jax.experimental.pallas import tpu as pltpu
from jax.experimental.pallas import tpu_sc as plsc

B, S, D_IN, D_OUT = 4, 32, 512, 512
N_ROWS = B * S
CAP = 96
BIG = 1e30
L = 16
NC, NS = 2, 16
TASKS_PER_SC = N_ROWS // NC
TASKS_PER_TILE = TASKS_PER_SC // NS
COLS_PER_TILE = D_OUT // NS


def _prep_body(w_ref, out_ref):
    out_ref[...] = jnp.full((8, 128), jnp.max(jnp.abs(w_ref[...])), jnp.float32)


def _wabs_max(W):
    return pl.pallas_call(
        _prep_body,
        out_shape=jax.ShapeDtypeStruct((8, 128), jnp.float32),
    )(W)


def _insert8(regs, v):
    out = []
    for r in regs:
        lo = jnp.minimum(r, v)
        v = jnp.maximum(r, v)
        out.append(lo)
    return out


def _shuffle(buf, v, idx):
    buf[...] = v
    return plsc.load_gather(buf, [idx])


def _sort16(buf, v):
    iota = lax.iota(jnp.int32, L)
    for k in (2, 4, 8, 16):
        j = k // 2
        while j >= 1:
            p = _shuffle(buf, v, iota ^ j)
            a_blk = (iota & k) == 0
            lower = (iota & j) == 0
            cond = a_blk == lower
            v = jnp.where(cond, jnp.minimum(v, p), jnp.maximum(v, p))
            j //= 2
    return v


def _prefix_sum16(buf, v):
    iota = lax.iota(jnp.int32, L)
    for d in (1, 2, 4, 8):
        sh = _shuffle(buf, v, jnp.maximum(iota - d, 0))
        v = v + jnp.where(iota >= d, sh, 0)
    return v


def _sum8(regs):
    s = regs[0]
    for r in regs[1:]:
        s = s + r
    return s * 0.125


def _sc_body(x_hbm, wt_hbm, stat_hbm, outp_hbm, outn_hbm,
             xv4, statv, sortv, ibuf, alist, slist, idxv,
             sh_a, sh_s, sh_i, sh_c,
             wtv, a2, s2, i2, c2, outpb, outnb, semw, semx):
    cid = lax.axis_index("c")
    sid = lax.axis_index("s")
    task_base = cid * TASKS_PER_SC + sid * TASKS_PER_TILE

    hw = pltpu.async_copy(
        wt_hbm.at[pl.ds(sid * COLS_PER_TILE * D_IN, COLS_PER_TILE * D_IN)],
        wtv, semw)
    hx = pltpu.async_copy(x_hbm.at[pl.ds(task_base, TASKS_PER_TILE)], xv4, semx)

    pltpu.sync_copy(stat_hbm, statv)
    spreadv = 2.0 * statv[...]
    iota = lax.iota(jnp.int32, L)
    idx7 = jnp.full((L,), 7, jnp.int32)
    idx15 = jnp.full((L,), 15, jnp.int32)
    zeros_i = jnp.zeros((L,), jnp.int32)
    bigv = jnp.full((L,), BIG, jnp.float32)
    onev = jnp.full((L,), 1.0, jnp.float32)

    hx.wait()

    for t in range(TASKS_PER_TILE):
        task_local = sid * TASKS_PER_TILE + t

        for k in range(CAP // L):
            idxv[pl.ds(k * L, L)] = zeros_i
            alist[pl.ds(k * L, L)] = bigv
            slist[pl.ds(k * L, L)] = onev

        def passa(j, lmin, t=t):
            xj = xv4[t, pl.ds(j * L, L)]
            lmin = jnp.minimum(lmin, jnp.maximum(3.0 + xj, 0.0))
            lmin = jnp.minimum(lmin, jnp.maximum(3.0 - xj, 0.0))
            return lmin

        lmin = plsc.parallel_loop(0, D_IN // L, unroll=2, carry=bigv)(passa)
        ks = _sort16(sortv, lmin)
        ub8 = _shuffle(sortv, ks, idx7)
        tauv = ub8 + spreadv

        def passb(j, offv, t=t):
            xj = xv4[t, pl.ds(j * L, L)]
            idx16 = iota + j * L
            for sgn in (1.0, -1.0):
                aval = jnp.maximum(3.0 + sgn * xj, 0.0)
                msk = aval <= tauv
                mi = msk.astype(jnp.int32)
                cs = _prefix_sum16(ibuf, mi)
                pos = offv + cs - mi
                okm = jnp.logical_and(msk, pos < CAP)
                plsc.store_scatter(alist, [pos], aval, mask=okm)
                plsc.store_scatter(slist, [pos],
                                   jnp.full((L,), sgn, jnp.float32), mask=okm)
                plsc.store_scatter(idxv, [pos], idx16, mask=okm)
                offv = offv + _shuffle(ibuf, cs, idx15)
            return offv

        offv = plsc.parallel_loop(0, D_IN // L, unroll=2, carry=zeros_i)(passb)
        ibuf[...] = jnp.minimum(offv, CAP)

        pltpu.sync_copy(alist, sh_a.at[pl.ds(task_local * CAP, CAP)])
        pltpu.sync_copy(slist, sh_s.at[pl.ds(task_local * CAP, CAP)])
        pltpu.sync_copy(idxv, sh_i.at[pl.ds(task_local * CAP, CAP)])
        pltpu.sync_copy(ibuf, sh_c.at[pl.ds(task_local * L, L)])

    plsc.subcore_barrier()

    pltpu.sync_copy(sh_a, a2)
    pltpu.sync_copy(sh_s, s2)
    pltpu.sync_copy(sh_i, i2)
    pltpu.sync_copy(sh_c, c2)
    hw.wait()

    colb0 = iota * D_IN
    colb1 = (iota + L) * D_IN

    def task_loop(tl, _):
        n_t = jnp.minimum(c2[pl.ds(tl * L, L)][0], CAP)

        def body(r, regs4):
            rsplat = jnp.broadcast_to(tl * CAP + r, (L,)).astype(jnp.int32)
            a16 = plsc.load_gather(a2, [rsplat])
            s16 = plsc.load_gather(s2, [rsplat])
            i16 = plsc.load_gather(i2, [rsplat])
            w0 = plsc.load_gather(wtv, [colb0 + i16])
            w1 = plsc.load_gather(wtv, [colb1 + i16])
            sw0 = s16 * w0
            sw1 = s16 * w1
            cp0 = a16 + jnp.maximum(3.0 + sw0, 0.0)
            cn0 = a16 + jnp.maximum(3.0 - sw0, 0.0)
            cp1 = a16 + jnp.maximum(3.0 + sw1, 0.0)
            cn1 = a16 + jnp.maximum(3.0 - sw1, 0.0)
            return (_insert8(regs4[0], cp0), _insert8(regs4[1], cn0),
                    _insert8(regs4[2], cp1), _insert8(regs4[3], cn1))

        init = ([bigv] * 8, [bigv] * 8, [bigv] * 8, [bigv] * 8)
        p0, n0, p1, n1 = plsc.parallel_loop(0, n_t, unroll=3, carry=init)(
            lambda r, regs4: body(r, regs4))
        outpb[tl, pl.ds(0, L)] = _sum8(p0)
        outnb[tl, pl.ds(0, L)] = _sum8(n0)
        outpb[tl, pl.ds(L, L)] = _sum8(p1)
        outnb[tl, pl.ds(L, L)] = _sum8(n1)
        return 0

    plsc.parallel_loop(0, TASKS_PER_SC)(lambda tl: task_loop(tl, 0))

    row0 = cid * TASKS_PER_SC
    cpad = sid * 128
    pltpu.sync_copy(outpb, outp_hbm.at[pl.ds(row0, TASKS_PER_SC),
                                       pl.ds(cpad, 128)])
    pltpu.sync_copy(outnb, outn_hbm.at[pl.ds(row0, TASKS_PER_SC),
                                       pl.ds(cpad, 128)])


def kernel(input, W):
    x2 = input.reshape(N_ROWS, D_IN)
    wt = W.T.reshape(-1)
    stat16 = _wabs_max(W)[0, :L]

    mesh = plsc.VectorSubcoreMesh(core_axis_name="c", subcore_axis_name="s",
                                  num_cores=NC)
    sc = functools.partial(
        pl.kernel,
        mesh=mesh,
        compiler_params=pltpu.CompilerParams(needs_layout_passes=False),
        out_type=[
            jax.ShapeDtypeStruct((N_ROWS, NS * 128), jnp.float32),
            jax.ShapeDtypeStruct((N_ROWS, NS * 128), jnp.float32),
        ],
        scratch_types=[
            pltpu.VMEM((TASKS_PER_TILE, D_IN), jnp.float32),
            pltpu.VMEM((L,), jnp.float32),
            pltpu.VMEM((L,), jnp.float32),
            pltpu.VMEM((L,), jnp.int32),
            pltpu.VMEM((CAP,), jnp.float32),
            pltpu.VMEM((CAP,), jnp.float32),
            pltpu.VMEM((CAP,), jnp.int32),
            pltpu.VMEM_SHARED((TASKS_PER_SC * CAP,), jnp.float32),
            pltpu.VMEM_SHARED((TASKS_PER_SC * CAP,), jnp.float32),
            pltpu.VMEM_SHARED((TASKS_PER_SC * CAP,), jnp.int32),
            pltpu.VMEM_SHARED((TASKS_PER_SC * L,), jnp.int32),
            pltpu.VMEM((COLS_PER_TILE * D_IN,), jnp.float32),
            pltpu.VMEM((TASKS_PER_SC * CAP,), jnp.float32),
            pltpu.VMEM((TASKS_PER_SC * CAP,), jnp.float32),
            pltpu.VMEM((TASKS_PER_SC * CAP,), jnp.int32),
            pltpu.VMEM((TASKS_PER_SC * L,), jnp.int32),
            pltpu.VMEM((TASKS_PER_SC, 128), jnp.float32),
            pltpu.VMEM((TASKS_PER_SC, 128), jnp.float32),
            pltpu.SemaphoreType.DMA,
            pltpu.SemaphoreType.DMA,
        ],
    )(_sc_body)
    outp, outn = sc(x2, wt, stat16)
    outp = outp.reshape(N_ROWS, NS, 128)[:, :, :COLS_PER_TILE]
    outn = outn.reshape(N_ROWS, NS, 128)[:, :, :COLS_PER_TILE]
    return outp.reshape(B, S, D_OUT), outn.reshape(B, S, D_OUT)


# speedup vs baseline: 1.0205x; 1.0205x over previous
import functools

import jax
import jax.numpy as jnp
from jax import lax
from jax.experimental import pallas as pl
from jax.experimental.pallas import tpu as pltpu
from jax.experimental.pallas import tpu_sc as plsc

B, S, D_IN, D_OUT = 4, 32, 512, 512
N_ROWS = B * S
CAP = 96
BIG = 1e30
L = 16
NC, NS = 2, 16
TASKS_PER_SC = N_ROWS // NC
TASKS_PER_TILE = TASKS_PER_SC // NS
COLS_PER_TILE = D_OUT // NS


def _prep_body(w_ref, out_ref):
    out_ref[...] = jnp.full((8, 128), jnp.max(jnp.abs(w_ref[...])), jnp.float32)


def _wabs_max(W):
    return pl.pallas_call(
        _prep_body,
        out_shape=jax.ShapeDtypeStruct((8, 128), jnp.float32),
    )(W)


def _insert8(regs, v):
    out = []
    for r in regs:
        lo = jnp.minimum(r, v)
        v = jnp.maximum(r, v)
        out.append(lo)
    return out


def _shuffle(buf, v, idx):
    buf[...] = v
    return plsc.load_gather(buf, [idx])


def _sort16(buf, v):
    iota = lax.iota(jnp.int32, L)
    for k in (2, 4, 8, 16):
        j = k // 2
        while j >= 1:
            p = _shuffle(buf, v, iota ^ j)
            a_blk = (iota & k) == 0
            lower = (iota & j) == 0
            cond = a_blk == lower
            v = jnp.where(cond, jnp.minimum(v, p), jnp.maximum(v, p))
            j //= 2
    return v


def _prefix_sum16(buf, v):
    iota = lax.iota(jnp.int32, L)
    for d in (1, 2, 4, 8):
        sh = _shuffle(buf, v, jnp.maximum(iota - d, 0))
        v = v + jnp.where(iota >= d, sh, 0)
    return v


def _sum8(regs):
    s = regs[0]
    for r in regs[1:]:
        s = s + r
    return s * 0.125


def _sc_body(x_hbm, wt_hbm, stat_hbm, outp_hbm, outn_hbm,
             xv4, statv, sortv, ibuf, alist, slist, idxv,
             sh_a, sh_s, sh_i, sh_c,
             wtv, a2, s2, i2, c2, outpb, outnb, semw, semx):
    cid = lax.axis_index("c")
    sid = lax.axis_index("s")
    task_base = cid * TASKS_PER_SC + sid * TASKS_PER_TILE

    hw = pltpu.async_copy(
        wt_hbm.at[pl.ds(sid * COLS_PER_TILE * D_IN, COLS_PER_TILE * D_IN)],
        wtv, semw)
    hx = pltpu.async_copy(x_hbm.at[pl.ds(task_base, TASKS_PER_TILE)], xv4, semx)

    pltpu.sync_copy(stat_hbm, statv)
    spreadv = 2.0 * statv[...]
    iota = lax.iota(jnp.int32, L)
    idx7 = jnp.full((L,), 7, jnp.int32)
    idx15 = jnp.full((L,), 15, jnp.int32)
    zeros_i = jnp.zeros((L,), jnp.int32)
    bigv = jnp.full((L,), BIG, jnp.float32)
    onev = jnp.full((L,), 1.0, jnp.float32)

    hx.wait()

    for t in range(TASKS_PER_TILE):
        task_local = sid * TASKS_PER_TILE + t

        for k in range(CAP // L):
            idxv[pl.ds(k * L, L)] = zeros_i
            alist[pl.ds(k * L, L)] = bigv
            slist[pl.ds(k * L, L)] = onev

        def passa(j, lmin, t=t):
            xj = xv4[t, pl.ds(j * L, L)]
            lmin = jnp.minimum(lmin, jnp.maximum(3.0 + xj, 0.0))
            lmin = jnp.minimum(lmin, jnp.maximum(3.0 - xj, 0.0))
            return lmin

        lmin = plsc.parallel_loop(0, D_IN // L, unroll=2, carry=bigv)(passa)
        ks = _sort16(sortv, lmin)
        ub8 = _shuffle(sortv, ks, idx7)
        tauv = ub8 + spreadv

        def passb(j, offv, t=t):
            xj = xv4[t, pl.ds(j * L, L)]
            idx16 = iota + j * L
            for sgn in (1.0, -1.0):
                aval = jnp.maximum(3.0 + sgn * xj, 0.0)
                msk = aval <= tauv
                mi = msk.astype(jnp.int32)
                cs = _prefix_sum16(ibuf, mi)
                pos = offv + cs - mi
                okm = jnp.logical_and(msk, pos < CAP)
                plsc.store_scatter(alist, [pos], aval, mask=okm)
                plsc.store_scatter(slist, [pos],
                                   jnp.full((L,), sgn, jnp.float32), mask=okm)
                plsc.store_scatter(idxv, [pos], idx16, mask=okm)
                offv = offv + _shuffle(ibuf, cs, idx15)
            return offv

        offv = plsc.parallel_loop(0, D_IN // L, unroll=2, carry=zeros_i)(passb)
        ibuf[...] = jnp.minimum(offv, CAP)

        pltpu.sync_copy(alist, sh_a.at[pl.ds(task_local * CAP, CAP)])
        pltpu.sync_copy(slist, sh_s.at[pl.ds(task_local * CAP, CAP)])
        pltpu.sync_copy(idxv, sh_i.at[pl.ds(task_local * CAP, CAP)])
        pltpu.sync_copy(ibuf, sh_c.at[pl.ds(task_local * L, L)])

    plsc.subcore_barrier()

    pltpu.sync_copy(sh_a, a2)
    pltpu.sync_copy(sh_s, s2)
    pltpu.sync_copy(sh_i, i2)
    pltpu.sync_copy(sh_c, c2)
    hw.wait()

    colb0 = iota * D_IN
    colb1 = (iota + L) * D_IN

    def task_loop(tl, _):
        n_t = jnp.minimum(c2[pl.ds(tl * L, L)][0], CAP)

        def body(r, regs4):
            rsplat = jnp.broadcast_to(tl * CAP + r, (L,)).astype(jnp.int32)
            a16 = plsc.load_gather(a2, [rsplat])
            s16 = plsc.load_gather(s2, [rsplat])
            i16 = plsc.load_gather(i2, [rsplat])
            w0 = plsc.load_gather(wtv, [colb0 + i16])
            w1 = plsc.load_gather(wtv, [colb1 + i16])
            sw0 = s16 * w0
            sw1 = s16 * w1
            cp0 = a16 + jnp.maximum(3.0 + sw0, 0.0)
            cn0 = a16 + jnp.maximum(3.0 - sw0, 0.0)
            cp1 = a16 + jnp.maximum(3.0 + sw1, 0.0)
            cn1 = a16 + jnp.maximum(3.0 - sw1, 0.0)
            return (_insert8(regs4[0], cp0), _insert8(regs4[1], cn0),
                    _insert8(regs4[2], cp1), _insert8(regs4[3], cn1))

        init = ([bigv] * 8, [bigv] * 8, [bigv] * 8, [bigv] * 8)
        p0, n0, p1, n1 = plsc.parallel_loop(0, n_t, unroll=2, carry=init)(
            lambda r, regs4: body(r, regs4))
        outpb[tl, pl.ds(0, L)] = _sum8(p0)
        outnb[tl, pl.ds(0, L)] = _sum8(n0)
        outpb[tl, pl.ds(L, L)] = _sum8(p1)
        outnb[tl, pl.ds(L, L)] = _sum8(n1)
        return 0

    plsc.parallel_loop(0, TASKS_PER_SC)(lambda tl: task_loop(tl, 0))

    row0 = cid * TASKS_PER_SC
    cpad = sid * 128
    pltpu.sync_copy(outpb, outp_hbm.at[pl.ds(row0, TASKS_PER_SC),
                                       pl.ds(cpad, 128)])
    pltpu.sync_copy(outnb, outn_hbm.at[pl.ds(row0, TASKS_PER_SC),
                                       pl.ds(cpad, 128)])


def kernel(input, W):
    x2 = input.reshape(N_ROWS, D_IN)
    wt = W.T.reshape(-1)
    stat16 = _wabs_max(W)[0, :L]

    mesh = plsc.VectorSubcoreMesh(core_axis_name="c", subcore_axis_name="s",
                                  num_cores=NC)
    sc = functools.partial(
        pl.kernel,
        mesh=mesh,
        compiler_params=pltpu.CompilerParams(needs_layout_passes=False),
        out_type=[
            jax.ShapeDtypeStruct((N_ROWS, NS * 128), jnp.float32),
            jax.ShapeDtypeStruct((N_ROWS, NS * 128), jnp.float32),
        ],
        scratch_types=[
            pltpu.VMEM((TASKS_PER_TILE, D_IN), jnp.float32),
            pltpu.VMEM((L,), jnp.float32),
            pltpu.VMEM((L,), jnp.float32),
            pltpu.VMEM((L,), jnp.int32),
            pltpu.VMEM((CAP,), jnp.float32),
            pltpu.VMEM((CAP,), jnp.float32),
            pltpu.VMEM((CAP,), jnp.int32),
            pltpu.VMEM_SHARED((TASKS_PER_SC * CAP,), jnp.float32),
            pltpu.VMEM_SHARED((TASKS_PER_SC * CAP,), jnp.float32),
            pltpu.VMEM_SHARED((TASKS_PER_SC * CAP,), jnp.int32),
            pltpu.VMEM_SHARED((TASKS_PER_SC * L,), jnp.int32),
            pltpu.VMEM((COLS_PER_TILE * D_IN,), jnp.float32),
            pltpu.VMEM((TASKS_PER_SC * CAP,), jnp.float32),
            pltpu.VMEM((TASKS_PER_SC * CAP,), jnp.float32),
            pltpu.VMEM((TASKS_PER_SC * CAP,), jnp.int32),
            pltpu.VMEM((TASKS_PER_SC * L,), jnp.int32),
            pltpu.VMEM((TASKS_PER_SC, 128), jnp.float32),
            pltpu.VMEM((TASKS_PER_SC, 128), jnp.float32),
            pltpu.SemaphoreType.DMA,
            pltpu.SemaphoreType.DMA,
        ],
    )(_sc_body)
    outp, outn = sc(x2, wt, stat16)
    outp = outp.reshape(N_ROWS, NS, 128)[:, :, :COLS_PER_TILE]
    outn = outn.reshape(N_ROWS, NS, 128)[:, :, :COLS_PER_TILE]
    return outp.reshape(B, S, D_OUT), outn.reshape(B, S, D_OUT)
